# baseline (speedup 1.0000x reference)
import jax
import jax.numpy as jnp
from jax import lax
from jax.experimental import pallas as pl
from jax.experimental.pallas import tpu as pltpu
from jax.experimental.pallas import tpu_sc as plsc

B = 16
C = 12288
CHUNK = 96
ITERS = C // CHUNK


def _body(coords_hbm, na_hbm, out_hbm, buf, na_v, out_v, *sems):
    c = lax.axis_index("c")
    s = lax.axis_index("s")

    @pl.when(s < 8)
    def _():
        b = c * 8 + s
        Q = C // 4
        cps = [pltpu.async_copy(
            coords_hbm.at[b, pl.ds(k * Q, Q)],
            buf.at[pl.ds(k * Q, Q)], sems[k]) for k in range(4)]
        pltpu.sync_copy(na_hbm, na_v)

        iota = lax.iota(jnp.int32, 16)
        nvec = na_v[...]
        bvec = jnp.zeros((16,), jnp.int32) + b
        n_vec = nvec.at[bvec].get(mode="promise_in_bounds")
        thr = 3 * n_vec
        def body(i, accs):
            base = i * CHUNK
            acc = list(accs)
            for j in range(CHUNK // 16):
                off = base + j * 16
                v = buf[pl.ds(off, 16)]
                m = (iota + off) < thr
                acc[j] = acc[j] + jnp.where(m, v, 0.0)
            return tuple(acc)

        zero = jnp.zeros((16,), jnp.float32)
        accs = (zero,) * 6
        QI = Q // CHUNK
        for k in range(4):
            cps[k].wait()
            accs = lax.fori_loop(k * QI, (k + 1) * QI, body, accs)
        accs = tuple(accs[j] + accs[j + 3] for j in range(3))

        out_v[...] = zero
        for j in range(3):
            comp = (iota + j * 16) % 3
            plsc.addupdate_scatter(out_v, [comp], accs[j])

        nf = n_vec.astype(jnp.float32)
        out_v[...] = out_v[...] / nf
        pltpu.sync_copy(out_v, out_hbm.at[b])


@jax.jit
def _center(input_coords, num_atoms):
    mesh = plsc.VectorSubcoreMesh(core_axis_name="c", subcore_axis_name="s")
    padded = pl.kernel(
        _body,
        mesh=mesh,
        out_type=jax.ShapeDtypeStruct((B, 16), jnp.float32),
        scratch_types=[
            pltpu.VMEM((C,), jnp.float32),
            pltpu.VMEM((16,), jnp.int32),
            pltpu.VMEM((16,), jnp.float32),
            pltpu.SemaphoreType.DMA,
            pltpu.SemaphoreType.DMA,
            pltpu.SemaphoreType.DMA,
            pltpu.SemaphoreType.DMA,
        ],
        compiler_params=pltpu.CompilerParams(needs_layout_passes=False),
    )(input_coords, num_atoms)
    return padded[:, :3]


def kernel(input_coords, num_atoms):
    return _center(input_coords, num_atoms.astype(jnp.int32))


# speedup vs baseline: 1.0195x; 1.0195x over previous
import jax
import jax.numpy as jnp
from jax import lax
from jax.experimental import pallas as pl
from jax.experimental.pallas import tpu as pltpu
from jax.experimental.pallas import tpu_sc as plsc

B = 16
C = 12288
HALF = C // 2
CHUNK = 96
ITERS = HALF // CHUNK


def _body(coords_hbm, na_hbm, out_hbm, buf, na_v, part_v, tmp_v, shared, sem):
    c = lax.axis_index("c")
    s = lax.axis_index("s")
    b = c * 8 + s // 2
    h = s % 2

    row_cp = pltpu.async_copy(
        coords_hbm.at[b, pl.ds(h * HALF, HALF)], buf, sem)
    pltpu.sync_copy(na_hbm, na_v)

    iota = lax.iota(jnp.int32, 16)
    nvec = na_v[...]
    bvec = jnp.zeros((16,), jnp.int32) + b
    n_vec = nvec.at[bvec].get(mode="promise_in_bounds")
    thr = 3 * n_vec - h * HALF
    valid = jnp.clip(thr[0], 0, HALF)
    full_iters = valid // CHUNK
    row_cp.wait()

    def body(i, accs):
        base = i * CHUNK
        acc = list(accs)
        for j in range(CHUNK // 16):
            acc[j] = acc[j] + buf[pl.ds(base + j * 16, 16)]
        return tuple(acc)

    zero = jnp.zeros((16,), jnp.float32)
    accs = lax.fori_loop(0, full_iters, body, (zero,) * 6)

    processed = full_iters * CHUNK
    base = jnp.minimum(full_iters, ITERS - 1) * CHUNK
    acc = list(accs)
    for j in range(CHUNK // 16):
        idx = iota + (base + j * 16)
        v = buf[pl.ds(base + j * 16, 16)]
        m = (idx < thr) & (idx >= processed)
        acc[j] = acc[j] + jnp.where(m, v, 0.0)
    accs = tuple(acc[j] + acc[j + 3] for j in range(3))

    part_v[...] = zero
    for j in range(3):
        comp = (iota + j * 16) % 3
        plsc.addupdate_scatter(part_v, [comp], accs[j])

    @pl.when(h == 1)
    def _():
        pltpu.sync_copy(part_v, shared.at[8 + c * 16 + s])

    plsc.subcore_barrier()

    @pl.when(h == 0)
    def _():
        pltpu.sync_copy(shared.at[8 + c * 16 + s + 1], tmp_v)
        nf = n_vec.astype(jnp.float32)
        part_v[...] = (part_v[...] + tmp_v[...]) / nf
        pltpu.sync_copy(part_v, out_hbm.at[c * 16 + s])


@jax.jit
def _center(input_coords, num_atoms):
    mesh = plsc.VectorSubcoreMesh(core_axis_name="c", subcore_axis_name="s")
    padded = pl.kernel(
        _body,
        mesh=mesh,
        out_type=jax.ShapeDtypeStruct((2 * B, 16), jnp.float32),
        scratch_types=[
            pltpu.VMEM((HALF,), jnp.float32),
            pltpu.VMEM((16,), jnp.int32),
            pltpu.VMEM((16,), jnp.float32),
            pltpu.VMEM((16,), jnp.float32),
            pltpu.VMEM_SHARED((40, 16), jnp.float32),
            pltpu.SemaphoreType.DMA,
        ],
        compiler_params=pltpu.CompilerParams(needs_layout_passes=False),
    )(input_coords, num_atoms)
    return padded[::2, :3]


def kernel(input_coords, num_atoms):
    return _center(input_coords, num_atoms.astype(jnp.int32))
